# Initial kernel scaffold; baseline (speedup 1.0000x reference)
import jax
import jax.numpy as jnp
from jax.experimental import pallas as pl


def kernel(ent_embs, node_id, edge_index, out_norm, rel_embs, e_r_bias, g_idx, weight_neighbor, W_ih, W_hh, b_ih, b_hh):
    raise NotImplementedError("write your pallas kernel here")


# speedup vs baseline: 2.7542x; 2.7542x over previous
import functools

import jax
import jax.numpy as jnp
from jax import lax
from jax.experimental import pallas as pl
from jax.experimental.pallas import tpu as pltpu
from jax.experimental.pallas import tpu_sc as plsc

_NC = 2
_NS = 16
_NW = _NC * _NS
_LANES = 128


def _sc_mesh():
    return plsc.VectorSubcoreMesh(core_axis_name="c", subcore_axis_name="s")


def _gather_h(ent_embs, nid_rows, np_rows, d):
    rows_per_tile = np_rows // _NW
    chunks = rows_per_tile // _LANES

    @functools.partial(
        pl.kernel,
        out_type=jax.ShapeDtypeStruct((np_rows, d), jnp.float32),
        mesh=_sc_mesh(),
        scratch_types=[
            pltpu.VMEM((chunks, _LANES), jnp.int32),
            pltpu.VMEM((_LANES, d), jnp.float32),
            pltpu.SemaphoreType.DMA,
        ],
    )
    def k(ent_hbm, nid_hbm, h_hbm, idx_v, rows_v, sem):
        c = lax.axis_index("c")
        s = lax.axis_index("s")
        wid = c * _NS + s
        pltpu.sync_copy(nid_hbm.at[wid], idx_v)
        for j in range(chunks):
            pltpu.async_copy(ent_hbm.at[idx_v.at[j]], rows_v, sem).wait()
            pltpu.sync_copy(
                rows_v, h_hbm.at[pl.ds(wid * rows_per_tile + j * _LANES, _LANES)]
            )

    return k(ent_embs, nid_rows)


def _edge_segsum(h, src_rows, dst_rows, zeros_block, acc_rows, d):
    e_rows_total = src_rows.shape[0]
    rows_per_tile = e_rows_total // _NW
    acc_per_sub = acc_rows // _NS

    @functools.partial(
        pl.kernel,
        out_type=jax.ShapeDtypeStruct((_NC, acc_rows, d), jnp.float32),
        mesh=_sc_mesh(),
        scratch_types=[
            pltpu.VMEM((rows_per_tile, _LANES), jnp.int32),
            pltpu.VMEM((rows_per_tile, _LANES), jnp.int32),
            pltpu.VMEM((_LANES, d), jnp.float32),
            pltpu.VMEM_SHARED((acc_rows, d), jnp.float32),
            pltpu.SemaphoreType.DMA,
        ],
    )
    def k(h_hbm, src_hbm, dst_hbm, zero_hbm, out_hbm, src_v, dst_v, rows_v,
          acc, sem):
        c = lax.axis_index("c")
        s = lax.axis_index("s")
        wid = c * _NS + s
        pltpu.sync_copy(zero_hbm, acc.at[pl.ds(s * acc_per_sub, acc_per_sub)])
        pltpu.sync_copy(src_hbm.at[pl.ds(wid * rows_per_tile, rows_per_tile)],
                        src_v)
        pltpu.sync_copy(dst_hbm.at[pl.ds(wid * rows_per_tile, rows_per_tile)],
                        dst_v)
        plsc.subcore_barrier()

        def body(j, carry):
            pltpu.async_copy(h_hbm.at[src_v.at[j]], rows_v, sem).wait()
            pltpu.sync_copy(rows_v, acc.at[dst_v.at[j]], add=True)
            return carry

        lax.fori_loop(0, rows_per_tile, body, 0)
        plsc.subcore_barrier()
        pltpu.sync_copy(acc.at[pl.ds(s * acc_per_sub, acc_per_sub)],
                        out_hbm.at[c, pl.ds(s * acc_per_sub, acc_per_sub)])

    return k(h, src_rows, dst_rows, zeros_block)


def _dense_body(p0, p1, er, onorm, wn, wt1, wt2, wht, bih, bhh, out):
    d = wn.shape[0]
    a = p0[...] + p1[...]
    t = jnp.dot(a, wn[...], preferred_element_type=jnp.float32)
    e_new = t * onorm[...]
    er_v = er[...]
    gi = (jnp.dot(e_new, wt1[...], preferred_element_type=jnp.float32)
          + jnp.dot(er_v, wt2[...], preferred_element_type=jnp.float32)
          + bih[...])
    gh = jnp.dot(er_v, wht[...], preferred_element_type=jnp.float32) + bhh[...]
    r = jax.nn.sigmoid(gi[:, :d] + gh[:, :d])
    z = jax.nn.sigmoid(gi[:, d:2 * d] + gh[:, d:2 * d])
    n = jnp.tanh(gi[:, 2 * d:] + r * gh[:, 2 * d:])
    h0 = (1.0 - z) * n + z * er_v
    h0 = jnp.maximum(h0, 0.0)
    norm = jnp.sqrt(jnp.sum(h0 * h0, axis=1, keepdims=True))
    out[...] = h0 / jnp.maximum(norm, 1e-12)


def _dense_tail(p0, p1, e_r_bias, out_norm, wn, wt1, wt2, wht, bih, bhh):
    n, d = e_r_bias.shape
    blk = 1000
    grid = n // blk
    row_spec = pl.BlockSpec((blk, d), lambda i: (i, 0))
    full = lambda a: pl.BlockSpec(a.shape, lambda i: (0,) * a.ndim)
    return pl.pallas_call(
        _dense_body,
        grid=(grid,),
        in_specs=[
            row_spec, row_spec, row_spec,
            pl.BlockSpec((blk, 1), lambda i: (i, 0)),
            full(wn), full(wt1), full(wt2), full(wht), full(bih), full(bhh),
        ],
        out_specs=row_spec,
        out_shape=jax.ShapeDtypeStruct((n, d), jnp.float32),
    )(p0, p1, e_r_bias, out_norm, wn, wt1, wt2, wht, bih, bhh)


def kernel(ent_embs, node_id, edge_index, out_norm, rel_embs, e_r_bias, g_idx,
           weight_neighbor, W_ih, W_hh, b_ih, b_hh):
    n, d = ent_embs.shape
    e = edge_index.shape[1]

    np_rows = ((n + _NW * _LANES - 1) // (_NW * _LANES)) * (_NW * _LANES)
    nid = jnp.concatenate(
        [node_id.astype(jnp.int32),
         jnp.zeros((np_rows - n,), jnp.int32)]).reshape(_NW, -1, _LANES)
    h = _gather_h(ent_embs, nid, np_rows, d)

    ep_quant = _NW * _LANES * 8
    ep = ((e + ep_quant - 1) // ep_quant) * ep_quant
    src = edge_index[0].astype(jnp.int32)
    dst = edge_index[1].astype(jnp.int32)
    src_rows = jnp.concatenate(
        [src, jnp.zeros((ep - e,), jnp.int32)]).reshape(-1, _LANES)
    dst_rows = jnp.concatenate(
        [dst, jnp.full((ep - e,), n, jnp.int32)]).reshape(-1, _LANES)

    acc_rows = ((n + 1 + _NS * 8 - 1) // (_NS * 8)) * (_NS * 8)
    zeros_block = jnp.zeros((acc_rows // _NS, d), jnp.float32)
    partials = _edge_segsum(h, src_rows, dst_rows, zeros_block, acc_rows, d)

    p0 = partials[0, :n]
    p1 = partials[1, :n]

    wt = W_ih.T
    out = _dense_tail(
        p0, p1, e_r_bias, out_norm,
        weight_neighbor, wt[:d], wt[d:], W_hh.T,
        b_ih.reshape(1, -1), b_hh.reshape(1, -1))
    return out


# speedup vs baseline: 3.0203x; 1.0966x over previous
import functools

import jax
import jax.numpy as jnp
from jax import lax
from jax.experimental import pallas as pl
from jax.experimental.pallas import tpu as pltpu
from jax.experimental.pallas import tpu_sc as plsc

_NC = 2
_NS = 16
_NW = _NC * _NS
_LANES = 128
_NBUF = 4


def _sc_mesh():
    return plsc.VectorSubcoreMesh(core_axis_name="c", subcore_axis_name="s")


_CHUNK = 64


def _edge_segsum(ent_embs, nid_pad, src_flat, dst_flat, zeros_block,
                 acc_rows, d):
    n_pad = nid_pad.shape[0]
    e_per_tile = src_flat.shape[0] // _NW
    n_chunks = e_per_tile // _CHUNK
    groups = n_chunks // _NBUF
    acc_per_sub = acc_rows // _NS

    @functools.partial(
        pl.kernel,
        out_type=jax.ShapeDtypeStruct((_NC, acc_rows, d), jnp.float32),
        mesh=_sc_mesh(),
        compiler_params=pltpu.CompilerParams(needs_layout_passes=False),
        scratch_types=[
            pltpu.VMEM((n_pad,), jnp.int32),
            [pltpu.VMEM((_CHUNK,), jnp.int32) for _ in range(_NBUF)],
            [pltpu.VMEM((_CHUNK,), jnp.int32) for _ in range(_NBUF)],
            [pltpu.VMEM((_CHUNK,), jnp.int32) for _ in range(_NBUF)],
            [pltpu.VMEM((_CHUNK, d), jnp.float32) for _ in range(_NBUF)],
            pltpu.VMEM_SHARED((acc_rows, d), jnp.float32),
            [pltpu.SemaphoreType.DMA for _ in range(_NBUF)],
            [pltpu.SemaphoreType.DMA for _ in range(_NBUF)],
        ],
    )
    def k(ent_hbm, nid_hbm, src_hbm, dst_hbm, zero_hbm, out_hbm,
          nid_v, srcb, dstb, cidxb, rows, acc, isems, rsems):
        c = lax.axis_index("c")
        s = lax.axis_index("s")
        wid = c * _NS + s
        ebase = wid * e_per_tile

        def idx_copies(j, b):
            off = ebase + j * _CHUNK
            a1 = pltpu.async_copy(src_hbm.at[pl.ds(off, _CHUNK)], srcb[b],
                                  isems[b])
            a2 = pltpu.async_copy(dst_hbm.at[pl.ds(off, _CHUNK)], dstb[b],
                                  isems[b])
            return a1, a2

        def start_idx(j, b):
            idx_copies(j, b)

        def wait_idx(j, b):
            a1, a2 = pltpu.make_async_copy(
                src_hbm.at[pl.ds(ebase + j * _CHUNK, _CHUNK)], srcb[b],
                isems[b]), pltpu.make_async_copy(
                dst_hbm.at[pl.ds(ebase + j * _CHUNK, _CHUNK)], dstb[b],
                isems[b])
            a1.wait()
            a2.wait()

        def fill_and_gather(b):
            for l in range(_CHUNK // 16):
                s16 = srcb[b][pl.ds(l * 16, 16)]
                cidxb[b][pl.ds(l * 16, 16)] = plsc.load_gather(nid_v, [s16])
            pltpu.async_copy(ent_hbm.at[cidxb[b]], rows[b], rsems[b])

        def wait_gather(b):
            pltpu.make_async_copy(ent_hbm.at[cidxb[b]], rows[b],
                                  rsems[b]).wait()

        pltpu.sync_copy(zero_hbm, acc.at[pl.ds(s * acc_per_sub, acc_per_sub)])
        pltpu.sync_copy(nid_hbm, nid_v)
        for b in range(_NBUF):
            start_idx(b, b)
        for b in range(_NBUF - 2):
            wait_idx(b, b)
            fill_and_gather(b)
        plsc.subcore_barrier()

        def group(g, carry):
            for b in range(_NBUF):
                j = g * _NBUF + b
                wait_gather(b)
                pltpu.sync_copy(rows[b], acc.at[dstb[b]], add=True)
                jn = j + (_NBUF - 2)
                bn = (b + _NBUF - 2) % _NBUF

                @pl.when(jn < n_chunks)
                def _():
                    wait_idx(jn, bn)
                    fill_and_gather(bn)
                jj = j + _NBUF

                @pl.when(jj < n_chunks)
                def _():
                    start_idx(jj, b)
            return carry

        lax.fori_loop(0, groups, group, 0)
        plsc.subcore_barrier()
        pltpu.sync_copy(acc.at[pl.ds(s * acc_per_sub, acc_per_sub)],
                        out_hbm.at[c, pl.ds(s * acc_per_sub, acc_per_sub)])

    return k(ent_embs, nid_pad, src_flat, dst_flat, zeros_block)


def _dense_body(p0, p1, er, onorm, wn, wt1, wt2, wht, bih, bhh, out):
    d = wn.shape[0]
    a = p0[...] + p1[...]
    t = jnp.dot(a, wn[...], preferred_element_type=jnp.float32)
    e_new = t * onorm[...]
    er_v = er[...]
    gi = (jnp.dot(e_new, wt1[...], preferred_element_type=jnp.float32)
          + jnp.dot(er_v, wt2[...], preferred_element_type=jnp.float32)
          + bih[...])
    gh = jnp.dot(er_v, wht[...], preferred_element_type=jnp.float32) + bhh[...]
    r = jax.nn.sigmoid(gi[:, :d] + gh[:, :d])
    z = jax.nn.sigmoid(gi[:, d:2 * d] + gh[:, d:2 * d])
    n = jnp.tanh(gi[:, 2 * d:] + r * gh[:, 2 * d:])
    h0 = (1.0 - z) * n + z * er_v
    h0 = jnp.maximum(h0, 0.0)
    norm = jnp.sqrt(jnp.sum(h0 * h0, axis=1, keepdims=True))
    out[...] = h0 / jnp.maximum(norm, 1e-12)


def _dense_tail(p0, p1, e_r_bias, out_norm, wn, wt1, wt2, wht, bih, bhh):
    n, d = e_r_bias.shape
    blk = 1000
    grid = n // blk
    row_spec = pl.BlockSpec((blk, d), lambda i: (i, 0))
    full = lambda a: pl.BlockSpec(a.shape, lambda i: (0,) * a.ndim)
    return pl.pallas_call(
        _dense_body,
        grid=(grid,),
        in_specs=[
            row_spec, row_spec, row_spec,
            pl.BlockSpec((blk, 1), lambda i: (i, 0)),
            full(wn), full(wt1), full(wt2), full(wht), full(bih), full(bhh),
        ],
        out_specs=row_spec,
        out_shape=jax.ShapeDtypeStruct((n, d), jnp.float32),
    )(p0, p1, e_r_bias, out_norm, wn, wt1, wt2, wht, bih, bhh)


def kernel(ent_embs, node_id, edge_index, out_norm, rel_embs, e_r_bias, g_idx,
           weight_neighbor, W_ih, W_hh, b_ih, b_hh):
    n, d = ent_embs.shape
    e = edge_index.shape[1]

    n_pad = ((n + 7) // 8) * 8
    nid = jnp.concatenate(
        [node_id.astype(jnp.int32), jnp.zeros((n_pad - n,), jnp.int32)])

    ep_quant = _NW * _LANES * _NBUF * 2
    ep = ((e + ep_quant - 1) // ep_quant) * ep_quant
    src = edge_index[0].astype(jnp.int32)
    dst = edge_index[1].astype(jnp.int32)
    src_flat = jnp.concatenate([src, jnp.zeros((ep - e,), jnp.int32)])
    dst_flat = jnp.concatenate([dst, jnp.full((ep - e,), n, jnp.int32)])

    acc_rows = ((n + 1 + _NS * 8 - 1) // (_NS * 8)) * (_NS * 8)
    zeros_block = jnp.zeros((acc_rows // _NS, d), jnp.float32)
    partials = _edge_segsum(ent_embs, nid, src_flat, dst_flat, zeros_block,
                            acc_rows, d)

    p0 = partials[0, :n]
    p1 = partials[1, :n]

    wt = W_ih.T
    out = _dense_tail(
        p0, p1, e_r_bias, out_norm,
        weight_neighbor, wt[:d], wt[d:], W_hh.T,
        b_ih.reshape(1, -1), b_hh.reshape(1, -1))
    return out
